# Initial kernel scaffold; baseline (speedup 1.0000x reference)
import functools

import jax
import jax.numpy as jnp
from jax.experimental import pallas as pl

N_PAD = 10240
BLK = 1024


def _dense_body(x2_ref, x1_ref, w_src_ref, u_ref, v_ref, h_ref, asrc_ref, adst_ref):
    x2 = x2_ref[...]
    h = jnp.dot(x2, w_src_ref[...], preferred_element_type=jnp.float32)
    h_ref[...] = h
    asrc_ref[...] = h @ u_ref[...]
    adst_ref[...] = x1_ref[...] @ v_ref[...]


def _dense(x2p, x1p, w_src, u, v):
    grid = N_PAD // BLK
    return pl.pallas_call(
        _dense_body,
        grid=(grid,),
        in_specs=[
            pl.BlockSpec((BLK, 128), lambda i: (i, 0)),
            pl.BlockSpec((BLK, 128), lambda i: (i, 0)),
            pl.BlockSpec((128, 128), lambda i: (0, 0)),
            pl.BlockSpec((128,), lambda i: (0,)),
            pl.BlockSpec((128,), lambda i: (0,)),
        ],
        out_specs=[
            pl.BlockSpec((BLK, 128), lambda i: (i, 0)),
            pl.BlockSpec((BLK,), lambda i: (i,)),
            pl.BlockSpec((BLK,), lambda i: (i,)),
        ],
        out_shape=[
            jax.ShapeDtypeStruct((N_PAD, 128), jnp.float32),
            jax.ShapeDtypeStruct((N_PAD,), jnp.float32),
            jax.ShapeDtypeStruct((N_PAD,), jnp.float32),
        ],
    )(x2p, x1p, w_src, u, v)


def kernel(pi_edge_index, slice1_X, slice2_X, W_src, W_dst, att_src, att_dst, bias):
    n_dst = slice1_X.shape[0]
    u = W_src @ att_src
    v = W_dst @ att_dst
    x2p = jnp.pad(slice2_X, ((0, N_PAD - slice2_X.shape[0]), (0, 0)))
    x1p = jnp.pad(slice1_X, ((0, N_PAD - slice1_X.shape[0]), (0, 0)))
    h_src, a_src, a_dst = _dense(x2p, x1p, W_src, u, v)
    h_src = h_src[: slice2_X.shape[0]]
    a_src = a_src[: slice2_X.shape[0]]
    a_dst = a_dst[:n_dst]

    src = pi_edge_index[0]
    dst = pi_edge_index[1]
    e = a_src[src] + a_dst[dst]
    e = jnp.where(e > 0, e, 0.2 * e)
    w = jnp.exp(e)
    denom = jax.ops.segment_sum(w, dst, num_segments=n_dst)
    acc = jax.ops.segment_sum(h_src[src] * w[:, None], dst, num_segments=n_dst)
    out = acc / (denom[:, None] + 1e-16) + bias
    return jax.nn.relu(out)


# speedup vs baseline: 1.9685x; 1.9685x over previous
import functools

import jax
import jax.numpy as jnp
from jax.experimental import pallas as pl

N_PAD = 10240
BLK = 1024


def _dense_body(x2_ref, x1_ref, w_src_ref, u_ref, v_ref, h_ref, asrc_ref, adst_ref):
    x2 = x2_ref[...]
    h = jnp.dot(x2, w_src_ref[...], preferred_element_type=jnp.float32)
    h_ref[...] = h
    asrc_ref[...] = h @ u_ref[...]
    adst_ref[...] = x1_ref[...] @ v_ref[...]


def _dense(x2p, x1p, w_src, u, v):
    grid = N_PAD // BLK
    return pl.pallas_call(
        _dense_body,
        grid=(grid,),
        in_specs=[
            pl.BlockSpec((BLK, 128), lambda i: (i, 0)),
            pl.BlockSpec((BLK, 128), lambda i: (i, 0)),
            pl.BlockSpec((128, 128), lambda i: (0, 0)),
            pl.BlockSpec((128,), lambda i: (0,)),
            pl.BlockSpec((128,), lambda i: (0,)),
        ],
        out_specs=[
            pl.BlockSpec((BLK, 128), lambda i: (i, 0)),
            pl.BlockSpec((BLK,), lambda i: (i,)),
            pl.BlockSpec((BLK,), lambda i: (i,)),
        ],
        out_shape=[
            jax.ShapeDtypeStruct((N_PAD, 128), jnp.float32),
            jax.ShapeDtypeStruct((N_PAD,), jnp.float32),
            jax.ShapeDtypeStruct((N_PAD,), jnp.float32),
        ],
    )(x2p, x1p, w_src, u, v)


def kernel(pi_edge_index, slice1_X, slice2_X, W_src, W_dst, att_src, att_dst, bias):
    n_dst = slice1_X.shape[0]
    u = att_src
    v = W_dst @ att_dst
    x2p = jnp.pad(slice2_X, ((0, N_PAD - slice2_X.shape[0]), (0, 0)))
    x1p = jnp.pad(slice1_X, ((0, N_PAD - slice1_X.shape[0]), (0, 0)))
    h_src, a_src, a_dst = _dense(x2p, x1p, W_src, u, v)
    h_src = h_src[: slice2_X.shape[0]]
    a_src = a_src[: slice2_X.shape[0]]
    a_dst = a_dst[:n_dst]

    src = pi_edge_index[0]
    dst = pi_edge_index[1]
    e = a_src[src] + a_dst[dst]
    e = jnp.where(e > 0, e, 0.2 * e)
    w = jnp.exp(e)
    denom = jax.ops.segment_sum(w, dst, num_segments=n_dst)
    acc = jax.ops.segment_sum(h_src[src] * w[:, None], dst, num_segments=n_dst)
    out = acc / (denom[:, None] + 1e-16) + bias
    return jax.nn.relu(out)


# speedup vs baseline: 18.7327x; 9.5164x over previous
import functools

import jax
import jax.numpy as jnp
from jax import lax
from jax.experimental import pallas as pl
from jax.experimental.pallas import tpu as pltpu
from jax.experimental.pallas import tpu_sc as plsc

N_PAD = 10240
BLK = 1024

NC = 2
NS = 16
EDGE_CHUNK = 1024
HALF = 512
IDX_ROWS = EDGE_CHUNK // 128


def _dense_body(x2_ref, x1_ref, w_src_ref, u_ref, v_ref, h_ref, asrc_ref, adst_ref):
    x2 = x2_ref[...]
    h = jnp.dot(x2, w_src_ref[...], preferred_element_type=jnp.float32)
    h_ref[...] = h
    asrc_ref[...] = h @ u_ref[...]
    adst_ref[...] = x1_ref[...] @ v_ref[...]


def _dense(x2p, x1p, w_src, u, v):
    grid = N_PAD // BLK
    return pl.pallas_call(
        _dense_body,
        grid=(grid,),
        in_specs=[
            pl.BlockSpec((BLK, 128), lambda i: (i, 0)),
            pl.BlockSpec((BLK, 128), lambda i: (i, 0)),
            pl.BlockSpec((128, 128), lambda i: (0, 0)),
            pl.BlockSpec((128,), lambda i: (0,)),
            pl.BlockSpec((128,), lambda i: (0,)),
        ],
        out_specs=[
            pl.BlockSpec((BLK, 128), lambda i: (i, 0)),
            pl.BlockSpec((BLK,), lambda i: (i,)),
            pl.BlockSpec((BLK,), lambda i: (i,)),
        ],
        out_shape=[
            jax.ShapeDtypeStruct((N_PAD, 128), jnp.float32),
            jax.ShapeDtypeStruct((N_PAD,), jnp.float32),
            jax.ShapeDtypeStruct((N_PAD,), jnp.float32),
        ],
    )(x2p, x1p, w_src, u, v)


def _make_edge_kernel(n_chunks):
    rows_per_tile = N_PAD // NS
    iters = n_chunks // (NC * NS)
    mesh = plsc.VectorSubcoreMesh(core_axis_name="c", subcore_axis_name="s")

    @functools.partial(
        pl.kernel,
        out_type=[
            jax.ShapeDtypeStruct((NC, N_PAD, 128), jnp.float32),
            jax.ShapeDtypeStruct((NC, N_PAD), jnp.float32),
        ],
        mesh=mesh,
        compiler_params=pltpu.CompilerParams(use_tc_tiling_on_sc=False,
                                             needs_layout_passes=False),
        scratch_types=[
            pltpu.VMEM((IDX_ROWS, 128), jnp.int32),
            pltpu.VMEM((IDX_ROWS, 128), jnp.int32),
            pltpu.VMEM((IDX_ROWS, 128), jnp.float32),
            pltpu.VMEM((IDX_ROWS, 128), jnp.float32),
            pltpu.VMEM((2 * 128, 128), jnp.float32),
            pltpu.VMEM((IDX_ROWS, 128), jnp.float32),
            pltpu.VMEM_SHARED((N_PAD, 128), jnp.float32),
            pltpu.VMEM_SHARED((N_PAD,), jnp.float32),
            pltpu.SemaphoreType.DMA,
            pltpu.SemaphoreType.DMA,
        ],
    )
    def edge_kernel(h_hbm, asrc_hbm, adst_hbm, src_hbm, dst_hbm, zrows_hbm,
                    zden_hbm, accp_hbm, denp_hbm,
                    sidx_v, didx_v, avs_v, avd_v, rows_v, w_v,
                    acc_s, den_s, sem_a, sem_r):
        c = lax.axis_index("c")
        s = lax.axis_index("s")
        wid = s * NC + c

        pltpu.sync_copy(zrows_hbm, acc_s.at[pl.ds(s * rows_per_tile, rows_per_tile)])

        @pl.when(s == 0)
        def _():
            pltpu.sync_copy(zden_hbm, den_s)

        plsc.subcore_barrier()

        def chunk_body(k, _):
            chunk_id = k * (NC * NS) + wid
            row0 = chunk_id * IDX_ROWS
            pltpu.sync_copy(src_hbm.at[pl.ds(row0, IDX_ROWS)], sidx_v)
            pltpu.sync_copy(dst_hbm.at[pl.ds(row0, IDX_ROWS)], didx_v)

            for j in range(IDX_ROWS):
                pltpu.async_copy(asrc_hbm.at[sidx_v.at[j]], avs_v.at[j], sem_a)
                pltpu.async_copy(adst_hbm.at[didx_v.at[j]], avd_v.at[j], sem_a)
            for j in range(IDX_ROWS):
                pltpu.make_async_copy(asrc_hbm.at[sidx_v.at[j]], avs_v.at[j],
                                      sem_a).wait()
                pltpu.make_async_copy(adst_hbm.at[didx_v.at[j]], avd_v.at[j],
                                      sem_a).wait()

            pltpu.async_copy(h_hbm.at[sidx_v.at[0]],
                             rows_v.at[pl.ds(0, 128)], sem_r)

            def w_body(g, _):
                r = g >> 3
                col = (g & 7) * 16
                e = avs_v[r, pl.ds(col, 16)] + avd_v[r, pl.ds(col, 16)]
                e = jnp.where(e > 0.0, e, 0.2 * e)
                w_v[r, pl.ds(col, 16)] = jnp.exp(e)
                return 0

            lax.fori_loop(0, EDGE_CHUNK // 16, w_body, 0)

            for j in range(IDX_ROWS):
                slot = j & 1
                pltpu.make_async_copy(h_hbm.at[sidx_v.at[j]],
                                      rows_v.at[pl.ds(slot * 128, 128)],
                                      sem_r).wait()
                if j + 1 < IDX_ROWS:
                    nslot = (j + 1) & 1
                    pltpu.async_copy(h_hbm.at[sidx_v.at[j + 1]],
                                     rows_v.at[pl.ds(nslot * 128, 128)], sem_r)

                def scale_body(r, _):
                    row = slot * 128 + r
                    wr = plsc.load_gather(
                        w_v, [jnp.full((16,), j, jnp.int32),
                              jnp.full((16,), r, jnp.int32)])
                    for jj in range(8):
                        rows_v[row, pl.ds(jj * 16, 16)] = (
                            rows_v[row, pl.ds(jj * 16, 16)] * wr)
                    return 0

                lax.fori_loop(0, 128, scale_body, 0)

                pltpu.sync_copy(rows_v.at[pl.ds(slot * 128, 128)],
                                acc_s.at[didx_v.at[j]], add=True)
                pltpu.sync_copy(w_v.at[j], den_s.at[didx_v.at[j]], add=True)

            return 0

        lax.fori_loop(0, iters, chunk_body, 0)

        plsc.subcore_barrier()

        pltpu.sync_copy(acc_s.at[pl.ds(s * rows_per_tile, rows_per_tile)],
                        accp_hbm.at[c, pl.ds(s * rows_per_tile, rows_per_tile)])

        @pl.when(s == 0)
        def _():
            pltpu.sync_copy(den_s, denp_hbm.at[c])

    return edge_kernel


def _combine_body(acc_ref, den_ref, bias_ref, out_ref):
    a = acc_ref[0] + acc_ref[1]
    d = den_ref[0, 0, 0] + den_ref[1, 0, 0]
    out_ref[...] = jax.nn.relu(a / (d[:, None] + 1e-16) + bias_ref[...][None, :])


def _combine(accp, denp, bias):
    grid = N_PAD // BLK
    den4 = denp.reshape(NC, grid, 1, BLK)
    return pl.pallas_call(
        _combine_body,
        grid=(grid,),
        in_specs=[
            pl.BlockSpec((NC, BLK, 128), lambda i: (0, i, 0)),
            pl.BlockSpec((NC, 1, 1, BLK), lambda i: (0, i, 0, 0)),
            pl.BlockSpec((128,), lambda i: (0,)),
        ],
        out_specs=pl.BlockSpec((BLK, 128), lambda i: (i, 0)),
        out_shape=jax.ShapeDtypeStruct((N_PAD, 128), jnp.float32),
    )(accp, den4, bias)


def kernel(pi_edge_index, slice1_X, slice2_X, W_src, W_dst, att_src, att_dst, bias):
    n_dst = slice1_X.shape[0]
    n_edges = pi_edge_index.shape[1]
    e_pad = ((n_edges + EDGE_CHUNK * NC * NS - 1)
             // (EDGE_CHUNK * NC * NS)) * (EDGE_CHUNK * NC * NS)

    v = W_dst @ att_dst
    x2p = jnp.pad(slice2_X, ((0, N_PAD - slice2_X.shape[0]), (0, 0)))
    x1p = jnp.pad(slice1_X, ((0, N_PAD - slice1_X.shape[0]), (0, 0)))
    h_src, a_src, a_dst = _dense(x2p, x1p, W_src, att_src, v)

    trash = N_PAD - 1
    src = pi_edge_index[0].astype(jnp.int32)
    dst = pi_edge_index[1].astype(jnp.int32)
    src2d = jnp.full((e_pad,), trash, jnp.int32).at[:n_edges].set(src).reshape(
        e_pad // 128, 128)
    dst2d = jnp.full((e_pad,), trash, jnp.int32).at[:n_edges].set(dst).reshape(
        e_pad // 128, 128)
    zrows = jnp.zeros((N_PAD // NS, 128), jnp.float32)
    zden = jnp.zeros((N_PAD,), jnp.float32)

    edge_kernel = _make_edge_kernel(e_pad // EDGE_CHUNK)
    accp, denp = edge_kernel(h_src, a_src, a_dst, src2d, dst2d, zrows, zden)

    return _combine(accp, denp, bias)[:n_dst]


# speedup vs baseline: 19.1152x; 1.0204x over previous
import functools

import jax
import jax.numpy as jnp
from jax import lax
from jax.experimental import pallas as pl
from jax.experimental.pallas import tpu as pltpu
from jax.experimental.pallas import tpu_sc as plsc

N_PAD = 10240
BLK = 1024

NC = 2
NS = 16
EDGE_CHUNK = 1024
HALF = 512
IDX_ROWS = EDGE_CHUNK // 128


def _dense_body(x2_ref, x1_ref, w_src_ref, u_ref, v_ref, h_ref, asrc_ref, adst_ref):
    x2 = x2_ref[...]
    h = jnp.dot(x2, w_src_ref[...], preferred_element_type=jnp.float32)
    h_ref[...] = h
    asrc_ref[...] = h @ u_ref[...]
    adst_ref[...] = x1_ref[...] @ v_ref[...]


def _dense(x2p, x1p, w_src, u, v):
    grid = N_PAD // BLK
    return pl.pallas_call(
        _dense_body,
        grid=(grid,),
        in_specs=[
            pl.BlockSpec((BLK, 128), lambda i: (i, 0)),
            pl.BlockSpec((BLK, 128), lambda i: (i, 0)),
            pl.BlockSpec((128, 128), lambda i: (0, 0)),
            pl.BlockSpec((128,), lambda i: (0,)),
            pl.BlockSpec((128,), lambda i: (0,)),
        ],
        out_specs=[
            pl.BlockSpec((BLK, 128), lambda i: (i, 0)),
            pl.BlockSpec((BLK,), lambda i: (i,)),
            pl.BlockSpec((BLK,), lambda i: (i,)),
        ],
        out_shape=[
            jax.ShapeDtypeStruct((N_PAD, 128), jnp.float32),
            jax.ShapeDtypeStruct((N_PAD,), jnp.float32),
            jax.ShapeDtypeStruct((N_PAD,), jnp.float32),
        ],
    )(x2p, x1p, w_src, u, v)


def _make_edge_kernel(n_chunks):
    rows_per_tile = N_PAD // NS
    iters = n_chunks // (NC * NS)
    mesh = plsc.VectorSubcoreMesh(core_axis_name="c", subcore_axis_name="s")

    @functools.partial(
        pl.kernel,
        out_type=[
            jax.ShapeDtypeStruct((NC, N_PAD, 128), jnp.float32),
            jax.ShapeDtypeStruct((NC, N_PAD), jnp.float32),
        ],
        mesh=mesh,
        compiler_params=pltpu.CompilerParams(use_tc_tiling_on_sc=False,
                                             needs_layout_passes=False),
        scratch_types=[
            pltpu.VMEM((IDX_ROWS, 128), jnp.int32),
            pltpu.VMEM((IDX_ROWS, 128), jnp.int32),
            pltpu.VMEM((IDX_ROWS, 128), jnp.float32),
            pltpu.VMEM((IDX_ROWS, 128), jnp.float32),
            pltpu.VMEM((2 * 128, 128), jnp.float32),
            pltpu.VMEM((IDX_ROWS, 128), jnp.float32),
            pltpu.VMEM_SHARED((N_PAD, 128), jnp.float32),
            pltpu.VMEM_SHARED((N_PAD,), jnp.float32),
            pltpu.SemaphoreType.DMA,
            pltpu.SemaphoreType.DMA,
            pltpu.SemaphoreType.DMA,
            pltpu.SemaphoreType.DMA,
        ],
    )
    def edge_kernel(h_hbm, asrc_hbm, adst_hbm, src_hbm, dst_hbm, zrows_hbm,
                    zden_hbm, accp_hbm, denp_hbm,
                    sidx_v, didx_v, avs_v, avd_v, rows_v, w_v,
                    acc_s, den_s, sem_a, sem_r, sem_s, sem_w):
        c = lax.axis_index("c")
        s = lax.axis_index("s")
        wid = s * NC + c

        pltpu.sync_copy(zrows_hbm, acc_s.at[pl.ds(s * rows_per_tile, rows_per_tile)])

        @pl.when(s == 0)
        def _():
            pltpu.sync_copy(zden_hbm, den_s)

        plsc.subcore_barrier()

        def chunk_body(k, _):
            chunk_id = k * (NC * NS) + wid
            row0 = chunk_id * IDX_ROWS
            pltpu.sync_copy(src_hbm.at[pl.ds(row0, IDX_ROWS)], sidx_v)
            pltpu.sync_copy(dst_hbm.at[pl.ds(row0, IDX_ROWS)], didx_v)

            for j in range(IDX_ROWS):
                pltpu.async_copy(asrc_hbm.at[sidx_v.at[j]], avs_v.at[j], sem_a)
                pltpu.async_copy(adst_hbm.at[didx_v.at[j]], avd_v.at[j], sem_a)
            for j in range(IDX_ROWS):
                pltpu.make_async_copy(asrc_hbm.at[sidx_v.at[j]], avs_v.at[j],
                                      sem_a).wait()
                pltpu.make_async_copy(adst_hbm.at[didx_v.at[j]], avd_v.at[j],
                                      sem_a).wait()

            pltpu.async_copy(h_hbm.at[sidx_v.at[0]],
                             rows_v.at[pl.ds(0, 128)], sem_r)

            def w_body(g, _):
                r = g >> 3
                col = (g & 7) * 16
                e = avs_v[r, pl.ds(col, 16)] + avd_v[r, pl.ds(col, 16)]
                e = jnp.where(e > 0.0, e, 0.2 * e)
                w_v[r, pl.ds(col, 16)] = jnp.exp(e)
                return 0

            lax.fori_loop(0, EDGE_CHUNK // 16, w_body, 0)

            def scatter_desc(j):
                return pltpu.make_async_copy(
                    rows_v.at[pl.ds((j & 1) * 128, 128)],
                    acc_s.at[didx_v.at[j]], sem_s)

            for j in range(IDX_ROWS):
                slot = j & 1
                pltpu.make_async_copy(h_hbm.at[sidx_v.at[j]],
                                      rows_v.at[pl.ds(slot * 128, 128)],
                                      sem_r).wait()

                @plsc.parallel_loop(0, 128, 1, unroll=4)
                def _(r):
                    row = slot * 128 + r
                    wr = plsc.load_gather(
                        w_v, [jnp.full((16,), j, jnp.int32),
                              jnp.full((16,), r, jnp.int32)])
                    for jj in range(8):
                        rows_v[row, pl.ds(jj * 16, 16)] = (
                            rows_v[row, pl.ds(jj * 16, 16)] * wr)

                if j >= 2:
                    scatter_desc(j - 2).wait()
                if j + 1 < IDX_ROWS:
                    nslot = (j + 1) & 1
                    pltpu.async_copy(h_hbm.at[sidx_v.at[j + 1]],
                                     rows_v.at[pl.ds(nslot * 128, 128)], sem_r)
                pltpu.async_copy(rows_v.at[pl.ds(slot * 128, 128)],
                                 acc_s.at[didx_v.at[j]], sem_s, add=True)
                pltpu.async_copy(w_v.at[j], den_s.at[didx_v.at[j]], sem_w,
                                 add=True)

            scatter_desc(IDX_ROWS - 2).wait()
            scatter_desc(IDX_ROWS - 1).wait()
            for j in range(IDX_ROWS):
                pltpu.make_async_copy(w_v.at[j], den_s.at[didx_v.at[j]],
                                      sem_w).wait()
            return 0

        lax.fori_loop(0, iters, chunk_body, 0)

        plsc.subcore_barrier()

        pltpu.sync_copy(acc_s.at[pl.ds(s * rows_per_tile, rows_per_tile)],
                        accp_hbm.at[c, pl.ds(s * rows_per_tile, rows_per_tile)])

        @pl.when(s == 0)
        def _():
            pltpu.sync_copy(den_s, denp_hbm.at[c])

    return edge_kernel


def _combine_body(acc_ref, den_ref, bias_ref, out_ref):
    a = acc_ref[0] + acc_ref[1]
    d = den_ref[0, 0, 0] + den_ref[1, 0, 0]
    out_ref[...] = jax.nn.relu(a / (d[:, None] + 1e-16) + bias_ref[...][None, :])


def _combine(accp, denp, bias):
    grid = N_PAD // BLK
    den4 = denp.reshape(NC, grid, 1, BLK)
    return pl.pallas_call(
        _combine_body,
        grid=(grid,),
        in_specs=[
            pl.BlockSpec((NC, BLK, 128), lambda i: (0, i, 0)),
            pl.BlockSpec((NC, 1, 1, BLK), lambda i: (0, i, 0, 0)),
            pl.BlockSpec((128,), lambda i: (0,)),
        ],
        out_specs=pl.BlockSpec((BLK, 128), lambda i: (i, 0)),
        out_shape=jax.ShapeDtypeStruct((N_PAD, 128), jnp.float32),
    )(accp, den4, bias)


def kernel(pi_edge_index, slice1_X, slice2_X, W_src, W_dst, att_src, att_dst, bias):
    n_dst = slice1_X.shape[0]
    n_edges = pi_edge_index.shape[1]
    e_pad = ((n_edges + EDGE_CHUNK * NC * NS - 1)
             // (EDGE_CHUNK * NC * NS)) * (EDGE_CHUNK * NC * NS)

    v = W_dst @ att_dst
    x2p = jnp.pad(slice2_X, ((0, N_PAD - slice2_X.shape[0]), (0, 0)))
    x1p = jnp.pad(slice1_X, ((0, N_PAD - slice1_X.shape[0]), (0, 0)))
    h_src, a_src, a_dst = _dense(x2p, x1p, W_src, att_src, v)

    trash = N_PAD - 1
    src = pi_edge_index[0].astype(jnp.int32)
    dst = pi_edge_index[1].astype(jnp.int32)
    src2d = jnp.full((e_pad,), trash, jnp.int32).at[:n_edges].set(src).reshape(
        e_pad // 128, 128)
    dst2d = jnp.full((e_pad,), trash, jnp.int32).at[:n_edges].set(dst).reshape(
        e_pad // 128, 128)
    zrows = jnp.zeros((N_PAD // NS, 128), jnp.float32)
    zden = jnp.zeros((N_PAD,), jnp.float32)

    edge_kernel = _make_edge_kernel(e_pad // EDGE_CHUNK)
    accp, denp = edge_kernel(h_src, a_src, a_dst, src2d, dst2d, zrows, zden)

    return _combine(accp, denp, bias)[:n_dst]
